# baseline (speedup 1.0000x reference)
import jax
import jax.numpy as jnp
from jax.experimental import pallas as pl
from jax.experimental.pallas import tpu as pltpu

SIZE = 32
V = SIZE * SIZE
C = 256
B = 64


def _fused_kernel(a_ref, x_ref, g_ref, lp_ref, idx_ref, lpv_ref):
    a = a_ref[...].reshape(1, C)
    x = x_ref[...].reshape(C, V)
    xc = jax.lax.dot_general(a, x, (((1,), (0,)), ((), ())))
    m = jnp.max(xc)
    lse = jnp.log(jnp.sum(jnp.exp(xc - m))) + m
    lp = xc - lse
    lp_ref[...] = lp.reshape(1, 1, V)
    s = lp + g_ref[...].reshape(1, V)
    smax = jnp.max(s)
    iota = jax.lax.broadcasted_iota(jnp.int32, (1, V), 1)
    idx = jnp.min(jnp.where(s == smax, iota, V))
    idx_ref[...] = idx.reshape(1, 1, 1)
    lpv_ref[...] = jnp.sum(jnp.where(iota == idx, lp, 0.0)).reshape(1, 1, 1)


def kernel(x, embedded_a):
    xr = x.reshape(B, C, V)
    ar = embedded_a.reshape(B, 1, C)
    g = jax.random.gumbel(jax.random.key(42), (B, 1, V), dtype=jnp.float32)
    lp, idx, lpv = pl.pallas_call(
        _fused_kernel,
        grid=(B,),
        in_specs=[
            pl.BlockSpec((1, 1, C), lambda b: (b, 0, 0)),
            pl.BlockSpec((1, C, V), lambda b: (b, 0, 0)),
            pl.BlockSpec((1, 1, V), lambda b: (b, 0, 0)),
        ],
        out_specs=[
            pl.BlockSpec((1, 1, V), lambda b: (b, 0, 0)),
            pl.BlockSpec((1, 1, 1), lambda b: (b, 0, 0)),
            pl.BlockSpec((1, 1, 1), lambda b: (b, 0, 0)),
        ],
        out_shape=[
            jax.ShapeDtypeStruct((B, 1, V), jnp.float32),
            jax.ShapeDtypeStruct((B, 1, 1), jnp.int32),
            jax.ShapeDtypeStruct((B, 1, 1), jnp.float32),
        ],
        compiler_params=pltpu.CompilerParams(
            dimension_semantics=("arbitrary",),
        ),
    )(ar, xr, g)
    idx = idx[:, 0, 0]
    arg_lst = jnp.stack([idx % SIZE, idx // SIZE], axis=-1)
    return (arg_lst, lpv[:, 0, 0], lp.reshape(B, V))
